# baseline (speedup 1.0000x reference)
import functools
import math

import jax
import jax.numpy as jnp
from jax import lax
from jax.experimental import pallas as pl
from jax.experimental.pallas import tpu as pltpu
from jax.experimental.pallas import tpu_sc as plsc

S = 2048
NH = 16
HD = 128
HID = 2048
E = 8
TOPK = 2
I = 1024
EPS = 1e-6

MT = 256
TILE = 256
NT_CAP = (S * TOPK + E * TILE) // TILE
CAP = NT_CAP * TILE
GB = 128


def _qkv_body(x_ref, w_ref, ln_ref, o_ref):
    x = x_ref[...]
    var = jnp.mean(x * x, axis=-1, keepdims=True)
    xn = x * jax.lax.rsqrt(var + EPS) * ln_ref[...]
    o_ref[...] = jnp.dot(xn, w_ref[...], preferred_element_type=jnp.float32)


def _qkv_mm(x, w, ln, nt=512):
    n = w.shape[1]
    return pl.pallas_call(
        _qkv_body,
        grid=(S // MT, n // nt),
        in_specs=[
            pl.BlockSpec((MT, HID), lambda i, j: (i, 0)),
            pl.BlockSpec((HID, nt), lambda i, j: (0, j)),
            pl.BlockSpec((1, HID), lambda i, j: (0, 0)),
        ],
        out_specs=pl.BlockSpec((MT, nt), lambda i, j: (i, j)),
        out_shape=jax.ShapeDtypeStruct((S, n), jnp.float32),
    )(x, w, ln)


def _rope_body(q_ref, k_ref, cos_ref, sin_ref, oq_ref, ok_ref):
    c = cos_ref[...][None]
    s = sin_ref[...][None]

    def rot(x):
        x1 = x[..., : HD // 2]
        x2 = x[..., HD // 2 :]
        return jnp.concatenate([-x2, x1], axis=-1)

    q = q_ref[...]
    k = k_ref[...]
    oq_ref[...] = q * c + rot(q) * s
    ok_ref[...] = k * c + rot(k) * s


def _rope(q3, k3, cos, sin):
    return pl.pallas_call(
        _rope_body,
        grid=(NH, S // MT),
        in_specs=[
            pl.BlockSpec((1, MT, HD), lambda h, i: (h, i, 0)),
            pl.BlockSpec((1, MT, HD), lambda h, i: (h, i, 0)),
            pl.BlockSpec((MT, HD), lambda h, i: (i, 0)),
            pl.BlockSpec((MT, HD), lambda h, i: (i, 0)),
        ],
        out_specs=[
            pl.BlockSpec((1, MT, HD), lambda h, i: (h, i, 0)),
            pl.BlockSpec((1, MT, HD), lambda h, i: (h, i, 0)),
        ],
        out_shape=[
            jax.ShapeDtypeStruct((NH, S, HD), jnp.float32),
            jax.ShapeDtypeStruct((NH, S, HD), jnp.float32),
        ],
    )(q3, k3, cos, sin)


def _flash_body(q_ref, k_ref, v_ref, o_ref):
    qt = pl.program_id(1)
    q = q_ref[0]
    scale = 1.0 / math.sqrt(HD)
    row = jax.lax.broadcasted_iota(jnp.int32, (MT, MT), 0) + qt * MT

    def body(i, carry):
        acc, m, l = carry
        k = k_ref[0, pl.ds(i * MT, MT), :]
        v = v_ref[0, pl.ds(i * MT, MT), :]
        s = jax.lax.dot_general(q, k, (((1,), (1,)), ((), ())),
                                preferred_element_type=jnp.float32) * scale
        col = jax.lax.broadcasted_iota(jnp.int32, (MT, MT), 1) + i * MT
        s = jnp.where(row >= col, s, -1e30)
        m_new = jnp.maximum(m, jnp.max(s, axis=1, keepdims=True))
        p = jnp.exp(s - m_new)
        alpha = jnp.exp(m - m_new)
        l = l * alpha + jnp.sum(p, axis=1, keepdims=True)
        acc = acc * alpha + jnp.dot(p, v, preferred_element_type=jnp.float32)
        return acc, m_new, l

    acc = jnp.zeros((MT, HD), jnp.float32)
    m0 = jnp.full((MT, 1), -1e30, jnp.float32)
    l0 = jnp.zeros((MT, 1), jnp.float32)
    acc, m, l = jax.lax.fori_loop(0, qt + 1, body, (acc, m0, l0))
    o_ref[0] = acc / l


def _flash(q3, k3, v3):
    return pl.pallas_call(
        _flash_body,
        grid=(NH, S // MT),
        in_specs=[
            pl.BlockSpec((1, MT, HD), lambda h, i: (h, i, 0)),
            pl.BlockSpec((1, S, HD), lambda h, i: (h, 0, 0)),
            pl.BlockSpec((1, S, HD), lambda h, i: (h, 0, 0)),
        ],
        out_specs=pl.BlockSpec((1, MT, HD), lambda h, i: (h, i, 0)),
        out_shape=jax.ShapeDtypeStruct((NH, S, HD), jnp.float32),
    )(q3, k3, v3)


def _oproj_body(a_ref, w_ref, r_ref, o_ref):
    o_ref[...] = r_ref[...] + jnp.dot(a_ref[...], w_ref[...],
                                      preferred_element_type=jnp.float32)


def _bf(x):
    return x.astype(jnp.bfloat16)


def _oproj(attn2, o_w, resid, nt=512):
    return pl.pallas_call(
        _oproj_body,
        grid=(S // MT, HID // nt),
        in_specs=[
            pl.BlockSpec((MT, HID), lambda i, j: (i, 0)),
            pl.BlockSpec((HID, nt), lambda i, j: (0, j)),
            pl.BlockSpec((MT, nt), lambda i, j: (i, j)),
        ],
        out_specs=pl.BlockSpec((MT, nt), lambda i, j: (i, j)),
        out_shape=jax.ShapeDtypeStruct((S, HID), jnp.float32),
    )(attn2, o_w, resid)


def _rms_body(x_ref, ln_ref, o_ref):
    x = x_ref[...]
    var = jnp.mean(x * x, axis=-1, keepdims=True)
    o_ref[...] = x * jax.lax.rsqrt(var + EPS) * ln_ref[...]


def _rmsnorm(x, ln):
    return pl.pallas_call(
        _rms_body,
        grid=(S // MT,),
        in_specs=[
            pl.BlockSpec((MT, HID), lambda i: (i, 0)),
            pl.BlockSpec((1, HID), lambda i: (0, 0)),
        ],
        out_specs=pl.BlockSpec((MT, HID), lambda i: (i, 0)),
        out_shape=jax.ShapeDtypeStruct((S, HID), jnp.float32),
    )(x, ln)


def _router_body(x_ref, gw_ref, w_ref, ids_ref, wf_ref):
    x = x_ref[...].astype(jnp.bfloat16).astype(jnp.float32)
    cols = [jnp.sum(x * gw_ref[pl.ds(e, 1), :].astype(jnp.bfloat16)
                    .astype(jnp.float32), axis=1, keepdims=True)
            for e in range(E)]
    cols.append(jnp.zeros((x.shape[0], 128 - E), jnp.float32))
    logits = jnp.concatenate(cols, axis=1)
    lane = jax.lax.broadcasted_iota(jnp.int32, logits.shape, 1)
    valid = lane < E
    logits = jnp.where(valid, logits, -1e30)
    mx = jnp.max(logits, axis=1, keepdims=True)
    ex = jnp.where(valid, jnp.exp(logits - mx), 0.0)
    p = ex / jnp.sum(ex, axis=1, keepdims=True)

    m1 = jnp.max(p, axis=1, keepdims=True)
    i1 = jnp.min(jnp.where(p >= m1, lane, 2 * E), axis=1, keepdims=True)
    p2 = jnp.where(lane == i1, -1.0, p)
    m2 = jnp.max(p2, axis=1, keepdims=True)
    i2 = jnp.min(jnp.where(p2 >= m2, lane, 2 * E), axis=1, keepdims=True)

    tot = m1 + m2
    w1 = m1 / tot
    w2 = m2 / tot
    w_ref[...] = jnp.concatenate([w1, w2], axis=1)
    ids_ref[...] = jnp.concatenate([i1, i2], axis=1)
    wf_ref[...] = jnp.where(lane == i1, w1, 0.0) + jnp.where(lane == i2, w2, 0.0)


def _router(x2, gate_pad):
    return pl.pallas_call(
        _router_body,
        grid=(S // MT,),
        in_specs=[
            pl.BlockSpec((MT, HID), lambda i: (i, 0)),
            pl.BlockSpec((E, HID), lambda i: (0, 0)),
        ],
        out_specs=[
            pl.BlockSpec((MT, TOPK), lambda i: (i, 0)),
            pl.BlockSpec((MT, TOPK), lambda i: (i, 0)),
            pl.BlockSpec((MT, 128), lambda i: (i, 0)),
        ],
        out_shape=[
            jax.ShapeDtypeStruct((S, TOPK), jnp.float32),
            jax.ShapeDtypeStruct((S, TOPK), jnp.int32),
            jax.ShapeDtypeStruct((S, 128), jnp.float32),
        ],
    )(x2, gate_pad)


def _cnt_body(wf_ref, cnt_ref):
    g = pl.program_id(0)
    oh = (wf_ref[...] > 0.0).astype(jnp.float32)
    s = jnp.sum(oh, axis=0, keepdims=True)

    @pl.when(g == 0)
    def _():
        cnt_ref[...] = s

    @pl.when(g > 0)
    def _():
        cnt_ref[...] += s


def _meta_counts(w_full):
    return pl.pallas_call(
        _cnt_body,
        grid=(S // GB,),
        in_specs=[pl.BlockSpec((GB, 128), lambda g: (g, 0))],
        out_specs=pl.BlockSpec((1, 128), lambda g: (0, 0)),
        out_shape=jax.ShapeDtypeStruct((1, 128), jnp.float32),
        compiler_params=pltpu.CompilerParams(
            dimension_semantics=("arbitrary",)),
    )(w_full)


def _dst_body(wf_ref, cnt_ref, dlo_ref, dhi_ref, wlo_ref, whi_ref, te_ref,
              base_ref):
    g = pl.program_id(0)
    lane = jax.lax.broadcasted_iota(jnp.int32, (GB, 128), 1)
    lane1 = jax.lax.broadcasted_iota(jnp.int32, (1, 128), 1)

    cnt = cnt_ref[...]
    pc = jnp.floor((cnt + (TILE - 1)) * (1.0 / TILE))
    offt = jnp.zeros((1, 128), jnp.float32)
    run = jnp.zeros((1, 1), jnp.float32)
    for e in range(E):
        offt = offt + jnp.where(lane1 == e, run, 0.0)
        run = run + jnp.sum(jnp.where(lane1 == e, pc, 0.0), axis=1,
                            keepdims=True)
    off_rows = offt * TILE

    @pl.when(g == 0)
    def _():
        base_ref[...] = jnp.zeros((1, 128), jnp.float32)

    wf = wf_ref[...]
    oh = (wf > 0.0).astype(jnp.float32)
    r = jax.lax.broadcasted_iota(jnp.int32, (GB, GB), 0)
    c = jax.lax.broadcasted_iota(jnp.int32, (GB, GB), 1)
    ltri = (c < r).astype(jnp.float32)
    local = jnp.dot(ltri, oh, preferred_element_type=jnp.float32)
    dest = off_rows + base_ref[...] + local

    big = jnp.int32(999)
    e_lo = jnp.min(jnp.where(oh > 0, lane, big), axis=1, keepdims=True)
    e_hi = jnp.max(jnp.where(oh > 0, lane, -1), axis=1, keepdims=True)
    dlo_ref[...] = jnp.sum(jnp.where(lane == e_lo, dest, 0.0), axis=1,
                           keepdims=True).astype(jnp.int32)
    dhi_ref[...] = jnp.sum(jnp.where(lane == e_hi, dest, 0.0), axis=1,
                           keepdims=True).astype(jnp.int32)
    wlo_ref[...] = jnp.sum(jnp.where(lane == e_lo, wf, 0.0), axis=1,
                           keepdims=True)
    whi_ref[...] = jnp.sum(jnp.where(lane == e_hi, wf, 0.0), axis=1,
                           keepdims=True)

    base_ref[...] += jnp.sum(oh, axis=0, keepdims=True)

    acc = jnp.zeros((1, 128), jnp.float32)
    for e in range(E):
        off_e = jnp.sum(jnp.where(lane1 == e, offt, 0.0), axis=1,
                        keepdims=True)
        acc = acc + (lane1.astype(jnp.float32) >= off_e).astype(jnp.float32)
    te_ref[...] = jnp.clip(acc - 1.0, 0, E - 1).astype(jnp.int32)


def _meta_dests(w_full, cnt):
    return pl.pallas_call(
        _dst_body,
        grid=(S // GB,),
        in_specs=[
            pl.BlockSpec((GB, 128), lambda g: (g, 0)),
            pl.BlockSpec((1, 128), lambda g: (0, 0)),
        ],
        out_specs=[
            pl.BlockSpec((GB, 1), lambda g: (g, 0)),
            pl.BlockSpec((GB, 1), lambda g: (g, 0)),
            pl.BlockSpec((GB, 1), lambda g: (g, 0)),
            pl.BlockSpec((GB, 1), lambda g: (g, 0)),
            pl.BlockSpec((1, 128), lambda g: (0, 0)),
        ],
        out_shape=[
            jax.ShapeDtypeStruct((S, 1), jnp.int32),
            jax.ShapeDtypeStruct((S, 1), jnp.int32),
            jax.ShapeDtypeStruct((S, 1), jnp.float32),
            jax.ShapeDtypeStruct((S, 1), jnp.float32),
            jax.ShapeDtypeStruct((1, 128), jnp.int32),
        ],
        scratch_shapes=[pltpu.VMEM((1, 128), jnp.float32)],
        compiler_params=pltpu.CompilerParams(
            dimension_semantics=("arbitrary",)),
    )(w_full, cnt)


_NCS = 2
_NSS = 16
_NW = _NCS * _NSS
_TPW = S // _NW
_CHUNK = 16


def _sc_dispatch(x2, dlo, dhi):
    mesh = plsc.VectorSubcoreMesh(core_axis_name="c", subcore_axis_name="s")

    @functools.partial(
        pl.kernel,
        out_type=jax.ShapeDtypeStruct((CAP, HID // 2), jnp.int32),
        mesh=mesh,
        scratch_types=[
            pltpu.VMEM((_CHUNK,), jnp.int32),
            pltpu.VMEM((_CHUNK,), jnp.int32),
            pltpu.VMEM((_CHUNK, HID // 2), jnp.int32),
            pltpu.SemaphoreType.DMA,
            pltpu.SemaphoreType.DMA,
        ],
    )
    def disp(x2_hbm, dlo_hbm, dhi_hbm, xs_hbm, ilo_v, ihi_v, rows_v, s1, s2):
        wid = lax.axis_index("s") * _NCS + lax.axis_index("c")
        for ci in range(_TPW // _CHUNK):
            base = wid * _TPW + ci * _CHUNK
            pltpu.sync_copy(dlo_hbm.at[pl.ds(base, _CHUNK)], ilo_v)
            pltpu.sync_copy(dhi_hbm.at[pl.ds(base, _CHUNK)], ihi_v)
            pltpu.sync_copy(x2_hbm.at[pl.ds(base, _CHUNK)], rows_v)
            a = pltpu.async_copy(rows_v, xs_hbm.at[ilo_v], s1)
            b = pltpu.async_copy(rows_v, xs_hbm.at[ihi_v], s2)
            a.wait()
            b.wait()

    return disp(x2, dlo, dhi)


def _sc_combine(y, dlo, dhi):
    mesh = plsc.VectorSubcoreMesh(core_axis_name="c", subcore_axis_name="s")

    @functools.partial(
        pl.kernel,
        out_type=(jax.ShapeDtypeStruct((S, HID), jnp.float32),
                  jax.ShapeDtypeStruct((S, HID), jnp.float32)),
        mesh=mesh,
        scratch_types=[
            pltpu.VMEM((_CHUNK,), jnp.int32),
            pltpu.VMEM((_CHUNK,), jnp.int32),
            pltpu.VMEM((_CHUNK, HID), jnp.float32),
            pltpu.VMEM((_CHUNK, HID), jnp.float32),
            pltpu.SemaphoreType.DMA,
            pltpu.SemaphoreType.DMA,
        ],
    )
    def comb(y_hbm, dlo_hbm, dhi_hbm, z0_hbm, z1_hbm, ilo_v, ihi_v,
             r0_v, r1_v, s1, s2):
        wid = lax.axis_index("s") * _NCS + lax.axis_index("c")
        for ci in range(_TPW // _CHUNK):
            base = wid * _TPW + ci * _CHUNK
            pltpu.sync_copy(dlo_hbm.at[pl.ds(base, _CHUNK)], ilo_v)
            pltpu.sync_copy(dhi_hbm.at[pl.ds(base, _CHUNK)], ihi_v)
            a = pltpu.async_copy(y_hbm.at[ilo_v], r0_v, s1)
            b = pltpu.async_copy(y_hbm.at[ihi_v], r1_v, s2)
            a.wait()
            b.wait()
            pltpu.sync_copy(r0_v, z0_hbm.at[pl.ds(base, _CHUNK)])
            pltpu.sync_copy(r1_v, z1_hbm.at[pl.ds(base, _CHUNK)])

    return comb(y, dlo, dhi)


def _gmm1_body(te_ref, xs_ref, wg_ref, wu_ref, h_ref):
    x = xs_ref[...]
    g = jnp.dot(x, wg_ref[0], preferred_element_type=jnp.float32)
    u = jnp.dot(x, wu_ref[0], preferred_element_type=jnp.float32)
    h_ref[...] = (g * jax.lax.logistic(g) * u).astype(jnp.bfloat16)


def _gmm1(xs, W13g, W13u, te):
    return pl.pallas_call(
        _gmm1_body,
        grid_spec=pltpu.PrefetchScalarGridSpec(
            num_scalar_prefetch=1,
            grid=(NT_CAP,),
            in_specs=[
                pl.BlockSpec((TILE, HID), lambda i, te: (i, 0)),
                pl.BlockSpec((1, HID, I), lambda i, te: (te[i], 0, 0)),
                pl.BlockSpec((1, HID, I), lambda i, te: (te[i], 0, 0)),
            ],
            out_specs=pl.BlockSpec((TILE, I), lambda i, te: (i, 0)),
        ),
        out_shape=jax.ShapeDtypeStruct((CAP, I), jnp.bfloat16),
        compiler_params=pltpu.CompilerParams(
            dimension_semantics=("arbitrary",)),
    )(te, xs, W13g, W13u)


def _gmm2_body(te_ref, h_ref, w2_ref, y_ref):
    y_ref[...] = jnp.dot(h_ref[...], w2_ref[0],
                         preferred_element_type=jnp.float32)


def _gmm2(h, W2, te):
    return pl.pallas_call(
        _gmm2_body,
        grid_spec=pltpu.PrefetchScalarGridSpec(
            num_scalar_prefetch=1,
            grid=(NT_CAP,),
            in_specs=[
                pl.BlockSpec((TILE, I), lambda i, te: (i, 0)),
                pl.BlockSpec((1, I, HID), lambda i, te: (te[i], 0, 0)),
            ],
            out_specs=pl.BlockSpec((TILE, HID), lambda i, te: (i, 0)),
        ),
        out_shape=jax.ShapeDtypeStruct((CAP, HID), jnp.float32),
        compiler_params=pltpu.CompilerParams(
            dimension_semantics=("arbitrary",)),
    )(te, h, W2)


def _shared_body(z0_ref, z1_ref, wlo_ref, whi_ref, r_ref, sg_ref, su_ref,
                 sd_ref, o_ref):
    moe = wlo_ref[...] * z0_ref[...] + whi_ref[...] * z1_ref[...]
    moeb = moe.astype(jnp.bfloat16)
    g = jnp.dot(moeb, sg_ref[...], preferred_element_type=jnp.float32)
    u = jnp.dot(moeb, su_ref[...], preferred_element_type=jnp.float32)
    h = (g * jax.lax.logistic(g) * u).astype(jnp.bfloat16)
    sh = jnp.dot(h, sd_ref[...], preferred_element_type=jnp.float32)
    o_ref[...] = r_ref[...] + moe + sh


def _shared(z0, z1, wlo, whi, resid2, sg_w, su_w, sd_w):
    return pl.pallas_call(
        _shared_body,
        grid=(S // MT,),
        in_specs=[
            pl.BlockSpec((MT, HID), lambda i: (i, 0)),
            pl.BlockSpec((MT, HID), lambda i: (i, 0)),
            pl.BlockSpec((MT, 1), lambda i: (i, 0)),
            pl.BlockSpec((MT, 1), lambda i: (i, 0)),
            pl.BlockSpec((MT, HID), lambda i: (i, 0)),
            pl.BlockSpec((HID, I), lambda i: (0, 0)),
            pl.BlockSpec((HID, I), lambda i: (0, 0)),
            pl.BlockSpec((I, HID), lambda i: (0, 0)),
        ],
        out_specs=pl.BlockSpec((MT, HID), lambda i: (i, 0)),
        out_shape=jax.ShapeDtypeStruct((S, HID), jnp.float32),
    )(z0, z1, wlo, whi, resid2, sg_w, su_w, sd_w)


def kernel(hidden_states, cos, sin, ln1_w, ln2_w, q_w, k_w, v_w, o_w,
           gate_w, W13, W2, sg_w, su_w, sd_w):
    flat = hidden_states.reshape(S, HID)
    ln1 = ln1_w.reshape(1, HID)
    ln2 = ln2_w.reshape(1, HID)

    w_qkv = jnp.concatenate([q_w, k_w, v_w], axis=1)
    qkv = _qkv_mm(flat, w_qkv, ln1)
    q3 = qkv[:, :HID].reshape(S, NH, HD).transpose(1, 0, 2)
    k3 = qkv[:, HID:2 * HID].reshape(S, NH, HD).transpose(1, 0, 2)
    v3 = qkv[:, 2 * HID:].reshape(S, NH, HD).transpose(1, 0, 2)

    q3, k3 = _rope(q3, k3, cos, sin)
    attn = _flash(q3, k3, v3)
    attn2 = attn.transpose(1, 0, 2).reshape(S, HID)

    hid1 = _oproj(attn2, o_w, flat)
    x2 = _rmsnorm(hid1, ln2)

    topk_w, topk_ids, w_full = _router(x2, gate_w)

    cnt = _meta_counts(w_full)
    dlo, dhi, wlo, whi, te = _meta_dests(w_full, cnt)
    dlo1 = dlo.reshape(S)
    dhi1 = dhi.reshape(S)
    x2p = jax.lax.bitcast_convert_type(
        _bf(x2).reshape(S, HID // 2, 2), jnp.int32)
    xsp = _sc_dispatch(x2p, dlo1, dhi1)
    xs = jax.lax.bitcast_convert_type(xsp, jnp.bfloat16).reshape(CAP, HID)
    h = _gmm1(xs, _bf(W13[:, :, :I]), _bf(W13[:, :, I:]), te.reshape(128))
    y = _gmm2(h, _bf(W2), te.reshape(128))
    z0, z1 = _sc_combine(y, dlo1, dhi1)
    out = _shared(z0, z1, wlo, whi, hid1, _bf(sg_w), _bf(su_w), _bf(sd_w))

    return out.reshape(1, S, HID), topk_w, topk_ids
